# baseline (speedup 1.0000x reference)
import jax
import jax.numpy as jnp
from jax.experimental import pallas as pl
from jax.experimental.pallas import tpu as pltpu

A = 16
P = 64
V = 1
D = 256
H = 128


def _body(author_ids, venue_ids, paper_ids, wa, wv, wp, wsf,
          emb_paper, emb_author, emb_venue,
          topic_vec, W1, b1, W2,
          out_ref,
          a_scr, v_scr, p_scr, t_scr, w1_scr, b1_scr, w2_scr,
          sem_auth, sem_pap):
    dense_copies = [
        pltpu.make_async_copy(W1, w1_scr, sem_auth),
        pltpu.make_async_copy(b1, b1_scr, sem_auth),
        pltpu.make_async_copy(W2, w2_scr, sem_auth),
        pltpu.make_async_copy(topic_vec, t_scr, sem_auth),
    ]
    auth_copies = [
        pltpu.make_async_copy(emb_author.at[author_ids[i]], a_scr.at[i], sem_auth)
        for i in range(A)
    ] + [
        pltpu.make_async_copy(emb_venue.at[venue_ids[0]], v_scr.at[0], sem_auth)
    ]
    pap_copies = [
        pltpu.make_async_copy(emb_paper.at[paper_ids[i]], p_scr.at[i], sem_pap)
        for i in range(P)
    ]
    for cp in auth_copies:
        cp.start()
    for cp in dense_copies:
        cp.start()
    for cp in pap_copies:
        cp.start()
    for cp in dense_copies:
        cp.wait()
    for cp in auth_copies:
        cp.wait()

    a = a_scr[...]
    h = jnp.maximum(
        jax.lax.dot_general(a, w1_scr[...], (((1,), (0,)), ((), ())),
                            preferred_element_type=jnp.float32) + b1_scr[...], 0.0)
    logits = jax.lax.dot_general(h, w2_scr[...], (((1,), (0,)), ((), ())),
                                 preferred_element_type=jnp.float32)
    m = jnp.max(logits)
    e = jnp.exp(logits - m)
    attn = e / jnp.sum(e)
    agg_author = jnp.sum(a * attn, axis=0, keepdims=True)

    w0, w1_, w2_, w3 = wa[0], wv[0], wp[0], wsf[0]
    wm = jnp.maximum(jnp.maximum(w0, w1_), jnp.maximum(w2_, w3))
    e0 = jnp.exp(w0 - wm)
    e1 = jnp.exp(w1_ - wm)
    e2 = jnp.exp(w2_ - wm)
    e3 = jnp.exp(w3 - wm)
    es = e0 + e1 + e2 + e3
    partial = (agg_author * (e0 / es) + v_scr[...] * (e1 / es)
               + t_scr[...] * (e3 / es))

    for cp in pap_copies:
        cp.wait()
    agg_paper = jnp.sum(p_scr[...], axis=0, keepdims=True) * (1.0 / P)
    out_ref[...] = partial + agg_paper * (e2 / es)


def kernel(emb_paper, emb_author, emb_venue, topic_vec, W1, b1, W2, b2,
           w_author, w_venue, w_paper, w_self,
           author_ids, venue_ids, paper_ids):
    any_spec = pl.BlockSpec(memory_space=pl.ANY)
    smem_spec = pl.BlockSpec(memory_space=pltpu.SMEM)
    grid_spec = pltpu.PrefetchScalarGridSpec(
        num_scalar_prefetch=3,
        grid=(1,),
        in_specs=[smem_spec] * 4 + [any_spec] * 7,
        out_specs=pl.BlockSpec((1, D), lambda *_: (0, 0)),
        scratch_shapes=[
            pltpu.VMEM((A, D), jnp.float32),
            pltpu.VMEM((V, D), jnp.float32),
            pltpu.VMEM((P, D), jnp.float32),
            pltpu.VMEM((1, D), jnp.float32),
            pltpu.VMEM((D, H), jnp.float32),
            pltpu.VMEM((1, H), jnp.float32),
            pltpu.VMEM((H, 1), jnp.float32),
            pltpu.SemaphoreType.DMA,
            pltpu.SemaphoreType.DMA,
        ],
    )
    out = pl.pallas_call(
        _body,
        grid_spec=grid_spec,
        out_shape=jax.ShapeDtypeStruct((1, D), jnp.float32),
    )(author_ids, venue_ids, paper_ids,
      w_author.reshape(1), w_venue.reshape(1),
      w_paper.reshape(1), w_self.reshape(1),
      emb_paper, emb_author, emb_venue,
      topic_vec.reshape(1, D), W1, b1.reshape(1, H), W2)
    return out.reshape(D)


# speedup vs baseline: 1.0350x; 1.0350x over previous
import jax
import jax.numpy as jnp
from jax.experimental import pallas as pl
from jax.experimental.pallas import tpu as pltpu

A = 16
P = 64
V = 1
D = 256
H = 128


def _body(author_ids, venue_ids, paper_ids, wa, wv, wp, wsf,
          emb_paper, emb_author, emb_venue,
          topic_vec, W1, b1, W2,
          out_ref,
          a_scr, v_scr, p_scr, t_scr, w1_scr, b1_scr, w2_scr, sem):
    copies = [
        pltpu.make_async_copy(W1, w1_scr, sem),
        pltpu.make_async_copy(b1, b1_scr, sem),
        pltpu.make_async_copy(W2, w2_scr, sem),
        pltpu.make_async_copy(topic_vec, t_scr, sem),
    ]
    for i in range(A):
        copies.append(pltpu.make_async_copy(
            emb_author.at[author_ids[i]], a_scr.at[i], sem))
    for i in range(V):
        copies.append(pltpu.make_async_copy(
            emb_venue.at[venue_ids[i]], v_scr.at[i], sem))
    for i in range(P):
        copies.append(pltpu.make_async_copy(
            emb_paper.at[paper_ids[i]], p_scr.at[i], sem))
    for cp in copies:
        cp.start()
    for cp in copies:
        cp.wait()

    a = a_scr[...]
    h = jnp.maximum(
        jax.lax.dot_general(a, w1_scr[...], (((1,), (0,)), ((), ())),
                            preferred_element_type=jnp.float32) + b1_scr[...], 0.0)
    logits = jax.lax.dot_general(h, w2_scr[...], (((1,), (0,)), ((), ())),
                                 preferred_element_type=jnp.float32)
    m = jnp.max(logits)
    e = jnp.exp(logits - m)
    attn = e / jnp.sum(e)
    agg_author = jnp.sum(a * attn, axis=0, keepdims=True)

    agg_venue = v_scr[...]
    agg_paper = jnp.sum(p_scr[...], axis=0, keepdims=True) * (1.0 / P)

    w0, w1_, w2_, w3 = wa[0], wv[0], wp[0], wsf[0]
    wm = jnp.maximum(jnp.maximum(w0, w1_), jnp.maximum(w2_, w3))
    e0 = jnp.exp(w0 - wm)
    e1 = jnp.exp(w1_ - wm)
    e2 = jnp.exp(w2_ - wm)
    e3 = jnp.exp(w3 - wm)
    es = e0 + e1 + e2 + e3
    out_ref[...] = (agg_author * (e0 / es) + agg_venue * (e1 / es)
                    + agg_paper * (e2 / es) + t_scr[...] * (e3 / es))


def kernel(emb_paper, emb_author, emb_venue, topic_vec, W1, b1, W2, b2,
           w_author, w_venue, w_paper, w_self,
           author_ids, venue_ids, paper_ids):
    any_spec = pl.BlockSpec(memory_space=pl.ANY)
    grid_spec = pltpu.PrefetchScalarGridSpec(
        num_scalar_prefetch=7,
        grid=(1,),
        in_specs=[any_spec] * 7,
        out_specs=pl.BlockSpec((1, D), lambda i, *_: (0, 0)),
        scratch_shapes=[
            pltpu.VMEM((A, D), jnp.float32),
            pltpu.VMEM((V, D), jnp.float32),
            pltpu.VMEM((P, D), jnp.float32),
            pltpu.VMEM((1, D), jnp.float32),
            pltpu.VMEM((D, H), jnp.float32),
            pltpu.VMEM((1, H), jnp.float32),
            pltpu.VMEM((H, 1), jnp.float32),
            pltpu.SemaphoreType.DMA,
        ],
    )
    out = pl.pallas_call(
        _body,
        grid_spec=grid_spec,
        out_shape=jax.ShapeDtypeStruct((1, D), jnp.float32),
    )(author_ids, venue_ids, paper_ids,
      w_author.reshape(1), w_venue.reshape(1),
      w_paper.reshape(1), w_self.reshape(1),
      emb_paper, emb_author, emb_venue,
      topic_vec.reshape(1, D), W1, b1.reshape(1, H), W2)
    return out.reshape(D)
